# Initial kernel scaffold; baseline (speedup 1.0000x reference)
import jax
import jax.numpy as jnp
from jax.experimental import pallas as pl


def kernel(x, edge_index, eps1, eps2, eps3, W1a, b1a, W1b, b1b, bn1_g, bn1_b, W2a, b2a, bn2_g, bn2_b, W3a, b3a, W3b, b3b, bn3_g, bn3_b, lin1_W, lin1_b, fc_W, fc_b):
    raise NotImplementedError("write your pallas kernel here")


# speedup vs baseline: 3.4633x; 3.4633x over previous
import functools

import jax
import jax.numpy as jnp
from jax import lax
from jax.experimental import pallas as pl
from jax.experimental.pallas import tpu as pltpu
from jax.experimental.pallas import tpu_sc as plsc

NC = 2
NS = 16
LANES = 16
NW = NC * NS

N = 50000
NRANGES = 4
R = 12544
RACC = R + 256
NPAD = NRANGES * R
WSPAN = R // NS
ZROWS = 200

BLK = 128
BE = 3200
NGROUPS = BE // LANES
STG = 160

ROWBLK = 512


def _cdiv(a, b):
    return -(-a // b)


def _pre_body(ce, nblk, cap,
              src_hbm, dst_hbm,
              counts_hbm, srcl_hbm, dstl_hbm,
              src_blk, dst_blk,
              st_s0, st_s1, st_s2, st_s3,
              st_d0, st_d1, st_d2, st_d3,
              cnt_v):
    c = lax.axis_index("c")
    s = lax.axis_index("s")
    w = s * NC + c
    base = w * ce
    st_s = (st_s0, st_s1, st_s2, st_s3)
    st_d = (st_d0, st_d1, st_d2, st_d3)
    iota = lax.iota(jnp.int32, LANES)

    def group_body(g, carry):
        curs, wrs = carry
        off = pl.multiple_of(g * LANES, LANES)
        dv = dst_blk[pl.ds(off, LANES)]
        sv = src_blk[pl.ds(off, LANES)]
        new_curs = []
        new_wrs = []
        for r in range(NRANGES):
            cur = curs[r]
            wr = wrs[r]
            m = (dv >= r * R) & (dv < (r + 1) * R)
            mi = m.astype(jnp.int32)
            pos = cur + plsc.cumsum(mi) - 1
            plsc.store_scatter(st_s[r], [pos], sv, mask=m)
            plsc.store_scatter(st_d[r], [pos], dv - (r * R), mask=m)
            cur = cur + jnp.sum(mi)

            def flush(args):
                cur_i, wr_i = args
                wrm = pl.multiple_of(wr_i, BLK)
                pltpu.sync_copy(st_s[r].at[pl.ds(0, BLK)],
                                srcl_hbm.at[w, r, pl.ds(wrm, BLK)])
                pltpu.sync_copy(st_d[r].at[pl.ds(0, BLK)],
                                dstl_hbm.at[w, r, pl.ds(wrm, BLK)])
                tail_s = st_s[r][pl.ds(BLK, LANES)]
                tail_d = st_d[r][pl.ds(BLK, LANES)]
                st_s[r][pl.ds(0, LANES)] = tail_s
                st_d[r][pl.ds(0, LANES)] = tail_d
                return cur_i - BLK, wr_i + BLK

            cur, wr = lax.cond(cur >= BLK, flush, lambda a: a, (cur, wr))
            new_curs.append(cur)
            new_wrs.append(wr)
        return tuple(new_curs), tuple(new_wrs)

    def blk_body(b, carry):
        boff = pl.multiple_of(base + b * BE, BE)
        pltpu.sync_copy(src_hbm.at[pl.ds(boff, BE)], src_blk)
        pltpu.sync_copy(dst_hbm.at[pl.ds(boff, BE)], dst_blk)
        return lax.fori_loop(0, NGROUPS, group_body, carry)

    zero4 = (jnp.int32(0),) * NRANGES
    curs, wrs = lax.fori_loop(0, nblk, blk_body, (zero4, zero4))

    cnt_vec = jnp.zeros((LANES,), jnp.int32)
    for r in range(NRANGES):
        cur = curs[r]
        wr = wrs[r]
        for g in range(BLK // LANES):
            lanes_g = g * LANES + iota
            keep = lanes_g < cur
            sv = st_s[r][pl.ds(g * LANES, LANES)]
            dvv = st_d[r][pl.ds(g * LANES, LANES)]
            st_s[r][pl.ds(g * LANES, LANES)] = jnp.where(keep, sv, 0)
            st_d[r][pl.ds(g * LANES, LANES)] = jnp.where(keep, dvv, R + iota)

        @pl.when(cur > 0)
        def _():
            wrm = pl.multiple_of(wr, BLK)
            pltpu.sync_copy(st_s[r].at[pl.ds(0, BLK)],
                            srcl_hbm.at[w, r, pl.ds(wrm, BLK)])
            pltpu.sync_copy(st_d[r].at[pl.ds(0, BLK)],
                            dstl_hbm.at[w, r, pl.ds(wrm, BLK)])

        total = wr + jnp.where(cur > 0, BLK, 0)
        cnt_vec = jnp.where(iota == r, total, cnt_vec)

    cnt_v[pl.ds(0, LANES)] = cnt_vec
    pltpu.sync_copy(cnt_v, counts_hbm.at[w])


def _preprocess(src_p, dst_p, ce, cap):
    nblk = ce // BE
    mesh = plsc.VectorSubcoreMesh(core_axis_name="c", subcore_axis_name="s",
                                  num_cores=NC, num_subcores=NS)
    out_type = [
        jax.ShapeDtypeStruct((NW, LANES), jnp.int32),
        jax.ShapeDtypeStruct((NW, NRANGES, cap), jnp.int32),
        jax.ShapeDtypeStruct((NW, NRANGES, cap), jnp.int32),
    ]
    scratch = ([pltpu.VMEM((BE,), jnp.int32)] * 2
               + [pltpu.VMEM((STG,), jnp.int32)] * 8
               + [pltpu.VMEM((LANES,), jnp.int32)])
    body = functools.partial(_pre_body, ce, nblk, cap)
    return pl.kernel(
        body, out_type=out_type, mesh=mesh, scratch_types=scratch,
        compiler_params=pltpu.CompilerParams(needs_layout_passes=False),
    )(src_p, dst_p)


def _agg_body(d, cap,
              h_hbm, counts_hbm, srcl_hbm, dstl_hbm, zeros_hbm,
              out_hbm,
              acc, cnts_v, idx_s, idx_d, rows, sem):
    c = lax.axis_index("c")
    s = lax.axis_index("s")
    pltpu.sync_copy(counts_hbm, cnts_v)

    for p in range(NRANGES // NC):
        r = c * (NRANGES // NC) + p

        for k in range(RACC // (NS * ZROWS)):
            off = pl.multiple_of(s * (RACC // NS) + k * ZROWS, 8)
            pltpu.sync_copy(zeros_hbm, acc.at[pl.ds(off, ZROWS)])
        plsc.subcore_barrier()

        iota = lax.iota(jnp.int32, LANES)
        for q in range(NW // NS):
            w2 = s * (NW // NS) + q
            crow = cnts_v[w2, pl.ds(0, LANES)]
            nb = jnp.sum(jnp.where(iota == r, crow, 0)) // BLK

            def blk_body(b, _):
                boff = pl.multiple_of(b * BLK, BLK)
                pltpu.sync_copy(srcl_hbm.at[w2, r, pl.ds(boff, BLK)], idx_s)
                pltpu.sync_copy(dstl_hbm.at[w2, r, pl.ds(boff, BLK)], idx_d)
                pltpu.async_copy(h_hbm.at[idx_s], rows, sem).wait()
                pltpu.sync_copy(rows, acc.at[idx_d], add=True)
                return 0

            lax.fori_loop(0, nb, blk_body, 0)
        plsc.subcore_barrier()

        off = pl.multiple_of(s * WSPAN, 8)
        goff = pl.multiple_of(r * R + s * WSPAN, 8)
        pltpu.sync_copy(acc.at[pl.ds(off, WSPAN)],
                        out_hbm.at[pl.ds(goff, WSPAN)])
        plsc.subcore_barrier()


def _aggregate(h, counts, srcl, dstl, zeros, d, cap):
    mesh = plsc.VectorSubcoreMesh(core_axis_name="c", subcore_axis_name="s",
                                  num_cores=NC, num_subcores=NS)
    out_type = jax.ShapeDtypeStruct((NPAD, d), jnp.float32)
    scratch = [
        pltpu.VMEM_SHARED((RACC, d), jnp.float32),
        pltpu.VMEM((NW, LANES), jnp.int32),
        pltpu.VMEM((BLK,), jnp.int32),
        pltpu.VMEM((BLK,), jnp.int32),
        pltpu.VMEM((BLK, d), jnp.float32),
        pltpu.SemaphoreType.DMA,
    ]
    body = functools.partial(_agg_body, d, cap)
    return pl.kernel(
        body, out_type=out_type, mesh=mesh, scratch_types=scratch,
        compiler_params=pltpu.CompilerParams(needs_layout_passes=False),
    )(h, counts, srcl, dstl, zeros)


def _dot(a, w):
    return jnp.dot(a.astype(jnp.bfloat16), w.astype(jnp.bfloat16),
                   preferred_element_type=jnp.float32)


def _mlp1_body(eps_ref, x_ref, agg_ref, w1a_ref, b1a_ref, w1b_ref, b1b_ref,
               s1_ref, t1_ref, out_ref):
    e = eps_ref[0, 0]
    z = (1.0 + e) * x_ref[...] + agg_ref[...][:, :64]
    h = jnp.maximum(_dot(z, w1a_ref[...]) + b1a_ref[...], 0.0)
    h = jnp.maximum(_dot(h, w1b_ref[...]) + b1b_ref[...], 0.0)
    out_ref[...] = h * s1_ref[...] + t1_ref[...]


def _mlp2_body(eps_ref, h_ref, agg_ref, w2a_ref, b2a_ref, s2_ref, t2_ref,
               out_ref):
    e = eps_ref[0, 0]
    z = (1.0 + e) * h_ref[...] + agg_ref[...]
    h = jnp.maximum(_dot(z, w2a_ref[...]) + b2a_ref[...], 0.0)
    out_ref[...] = h * s2_ref[...] + t2_ref[...]


def _mlp3_body(eps_ref, h2_ref, agg_ref, w3a_ref, b3a_ref, w3b_ref, b3b_ref,
               s3_ref, t3_ref, l1w_ref, l1b_ref, fcw_ref, fcb_ref, out_ref):
    e = eps_ref[0, 0]
    z = (1.0 + e) * h2_ref[...] + agg_ref[...]
    a = jnp.maximum(_dot(z, w3a_ref[...]) + b3a_ref[...], 0.0)
    a = jnp.maximum(_dot(a, w3b_ref[...]) + b3b_ref[...], 0.0)
    a = a * s3_ref[...] + t3_ref[...]
    d = jnp.maximum(_dot(a, l1w_ref[...]) + l1b_ref[...], 0.0)
    db = d.astype(jnp.bfloat16).astype(jnp.float32)
    fb = fcw_ref[...].astype(jnp.bfloat16).astype(jnp.float32)
    o = jnp.sum(db * fb, axis=1) + fcb_ref[0, 0]
    out_ref[...] = jnp.broadcast_to(o[None, :], (8, ROWBLK))


def _row_spec(d):
    return pl.BlockSpec((ROWBLK, d), lambda i: (i, 0))


def _full_spec(shape):
    return pl.BlockSpec(shape, lambda i: (0,) * len(shape))


def _mlp1(eps1, x, agg0, w1a, b1a, w1b, b1b, s1, t1):
    grid = (NPAD // ROWBLK,)
    return pl.pallas_call(
        _mlp1_body,
        grid=grid,
        in_specs=[_full_spec((1, 1)), _row_spec(64), _row_spec(128),
                  _full_spec((64, 128)), _full_spec((1, 128)),
                  _full_spec((128, 128)), _full_spec((1, 128)),
                  _full_spec((1, 128)), _full_spec((1, 128))],
        out_specs=_row_spec(128),
        out_shape=jax.ShapeDtypeStruct((NPAD, 128), jnp.float32),
    )(eps1, x, agg0, w1a, b1a, w1b, b1b, s1, t1)


def _mlp2(eps2, h, agg1, w2a, b2a, s2, t2):
    grid = (NPAD // ROWBLK,)
    return pl.pallas_call(
        _mlp2_body,
        grid=grid,
        in_specs=[_full_spec((1, 1)), _row_spec(128), _row_spec(128),
                  _full_spec((128, 128)), _full_spec((1, 128)),
                  _full_spec((1, 128)), _full_spec((1, 128))],
        out_specs=_row_spec(128),
        out_shape=jax.ShapeDtypeStruct((NPAD, 128), jnp.float32),
    )(eps2, h, agg1, w2a, b2a, s2, t2)


def _mlp3(eps3, h2, agg2, w3a, b3a, w3b, b3b, s3, t3, l1w, l1b, fcw, fcb):
    grid = (NPAD // ROWBLK,)
    return pl.pallas_call(
        _mlp3_body,
        grid=grid,
        in_specs=[_full_spec((1, 1)), _row_spec(128), _row_spec(128),
                  _full_spec((128, 128)), _full_spec((1, 128)),
                  _full_spec((128, 128)), _full_spec((1, 128)),
                  _full_spec((1, 128)), _full_spec((1, 128)),
                  _full_spec((128, 128)), _full_spec((1, 128)),
                  _full_spec((1, 128)), _full_spec((1, 1))],
        out_specs=pl.BlockSpec((8, ROWBLK), lambda i: (0, i)),
        out_shape=jax.ShapeDtypeStruct((8, NPAD), jnp.float32),
    )(eps3, h2, agg2, w3a, b3a, w3b, b3b, s3, t3, l1w, l1b, fcw, fcb)


def kernel(x, edge_index, eps1, eps2, eps3, W1a, b1a, W1b, b1b, bn1_g, bn1_b,
           W2a, b2a, bn2_g, bn2_b, W3a, b3a, W3b, b3b, bn3_g, bn3_b,
           lin1_W, lin1_b, fc_W, fc_b):
    n, din = x.shape
    e = edge_index.shape[1]
    ce = _cdiv(_cdiv(e, NW), BE) * BE
    epad = ce * NW
    cap = ce

    src = edge_index[0]
    dst = edge_index[1]
    pad = epad - e
    src_p = jnp.concatenate([src, jnp.zeros((pad,), jnp.int32)])
    dst_p = jnp.concatenate([dst, jnp.full((pad,), jnp.int32(1 << 20))])
    x_p = jnp.pad(x, ((0, NPAD - n), (0, 0)))
    x_wide = jnp.pad(x, ((0, NPAD - n), (0, 128 - din)))

    counts, srcl, dstl = _preprocess(src_p, dst_p, ce, cap)

    inv = 1.0 / jnp.sqrt(1.0 + 1e-5)
    s1 = (bn1_g * inv)[None, :]; t1 = bn1_b[None, :]
    s2 = (bn2_g * inv)[None, :]; t2 = bn2_b[None, :]
    s3 = (bn3_g * inv)[None, :]; t3 = bn3_b[None, :]
    e1 = eps1.reshape(1, 1); e2 = eps2.reshape(1, 1); e3 = eps3.reshape(1, 1)

    zeros128 = jnp.zeros((ZROWS, 128), jnp.float32)
    agg0 = _aggregate(x_wide, counts, srcl, dstl, zeros128, 128, cap)
    h = _mlp1(e1, x_p, agg0, W1a, b1a[None, :], W1b, b1b[None, :], s1, t1)
    agg1 = _aggregate(h, counts, srcl, dstl, zeros128, 128, cap)
    h2 = _mlp2(e2, h, agg1, W2a, b2a[None, :], s2, t2)
    agg2 = _aggregate(h2, counts, srcl, dstl, zeros128, 128, cap)
    out2d = _mlp3(e3, h2, agg2, W3a, b3a[None, :], W3b, b3b[None, :], s3, t3,
                  lin1_W, lin1_b[None, :], fc_W.reshape(1, -1),
                  fc_b.reshape(1, 1))
    return out2d[0, :n]


# speedup vs baseline: 3.7466x; 1.0818x over previous
import functools

import jax
import jax.numpy as jnp
from jax import lax
from jax.experimental import pallas as pl
from jax.experimental.pallas import tpu as pltpu
from jax.experimental.pallas import tpu_sc as plsc

NC = 2
NS = 16
LANES = 16
NW = NC * NS

N = 50000
NRANGES = 6
R = 8704
RACC = R + 256
NPAD = NRANGES * R
WSPAN = R // NS
ZROWS = 280

BLK = 128
BE = 3200
NGROUPS = BE // LANES
STG = BLK + LANES

ROWBLK = 512


def _cdiv(a, b):
    return -(-a // b)


def _pre_body(ce, nblk, cap,
              src_hbm, dst_hbm,
              counts_hbm, srcl_hbm, dstl_hbm,
              src_blk, dst_blk, cnt_v, *stages):
    c = lax.axis_index("c")
    s = lax.axis_index("s")
    w = s * NC + c
    base = w * ce
    st_s = stages[:NRANGES]
    st_d = stages[NRANGES:]
    iota = lax.iota(jnp.int32, LANES)

    def group_body(g, carry):
        curs, wrs = carry
        off = pl.multiple_of(g * LANES, LANES)
        dv = dst_blk[pl.ds(off, LANES)]
        sv = src_blk[pl.ds(off, LANES)]
        new_curs = []
        new_wrs = []
        for r in range(NRANGES):
            cur = curs[r]
            wr = wrs[r]
            m = (dv >= r * R) & (dv < (r + 1) * R)
            mi = m.astype(jnp.int32)
            pos = cur + plsc.cumsum(mi) - 1
            plsc.store_scatter(st_s[r], [pos], sv, mask=m)
            plsc.store_scatter(st_d[r], [pos], dv - (r * R), mask=m)
            cur = cur + jnp.sum(mi)

            def flush(args):
                cur_i, wr_i = args
                wrm = pl.multiple_of(wr_i, BLK)
                pltpu.sync_copy(st_s[r].at[pl.ds(0, BLK)],
                                srcl_hbm.at[w, r, pl.ds(wrm, BLK)])
                pltpu.sync_copy(st_d[r].at[pl.ds(0, BLK)],
                                dstl_hbm.at[w, r, pl.ds(wrm, BLK)])
                tail_s = st_s[r][pl.ds(BLK, LANES)]
                tail_d = st_d[r][pl.ds(BLK, LANES)]
                st_s[r][pl.ds(0, LANES)] = tail_s
                st_d[r][pl.ds(0, LANES)] = tail_d
                return cur_i - BLK, wr_i + BLK

            cur, wr = lax.cond(cur >= BLK, flush, lambda a: a, (cur, wr))
            new_curs.append(cur)
            new_wrs.append(wr)
        return tuple(new_curs), tuple(new_wrs)

    def blk_body(b, carry):
        boff = pl.multiple_of(base + b * BE, BE)
        pltpu.sync_copy(src_hbm.at[pl.ds(boff, BE)], src_blk)
        pltpu.sync_copy(dst_hbm.at[pl.ds(boff, BE)], dst_blk)
        return lax.fori_loop(0, NGROUPS, group_body, carry)

    zero4 = (jnp.int32(0),) * NRANGES
    curs, wrs = lax.fori_loop(0, nblk, blk_body, (zero4, zero4))

    cnt_vec = jnp.zeros((LANES,), jnp.int32)
    for r in range(NRANGES):
        cur = curs[r]
        wr = wrs[r]
        for g in range(BLK // LANES):
            lanes_g = g * LANES + iota
            keep = lanes_g < cur
            sv = st_s[r][pl.ds(g * LANES, LANES)]
            dvv = st_d[r][pl.ds(g * LANES, LANES)]
            st_s[r][pl.ds(g * LANES, LANES)] = jnp.where(keep, sv, 0)
            st_d[r][pl.ds(g * LANES, LANES)] = jnp.where(keep, dvv, R + iota)

        @pl.when(cur > 0)
        def _():
            wrm = pl.multiple_of(wr, BLK)
            pltpu.sync_copy(st_s[r].at[pl.ds(0, BLK)],
                            srcl_hbm.at[w, r, pl.ds(wrm, BLK)])
            pltpu.sync_copy(st_d[r].at[pl.ds(0, BLK)],
                            dstl_hbm.at[w, r, pl.ds(wrm, BLK)])

        total = wr + jnp.where(cur > 0, BLK, 0)
        cnt_vec = jnp.where(iota == r, total, cnt_vec)

    cnt_v[pl.ds(0, LANES)] = cnt_vec
    pltpu.sync_copy(cnt_v, counts_hbm.at[w])


def _preprocess(src_p, dst_p, ce, cap):
    nblk = ce // BE
    mesh = plsc.VectorSubcoreMesh(core_axis_name="c", subcore_axis_name="s",
                                  num_cores=NC, num_subcores=NS)
    out_type = [
        jax.ShapeDtypeStruct((NW, LANES), jnp.int32),
        jax.ShapeDtypeStruct((NW, NRANGES, cap), jnp.int32),
        jax.ShapeDtypeStruct((NW, NRANGES, cap), jnp.int32),
    ]
    scratch = ([pltpu.VMEM((BE,), jnp.int32)] * 2
               + [pltpu.VMEM((LANES,), jnp.int32)]
               + [pltpu.VMEM((STG,), jnp.int32)] * (2 * NRANGES))
    body = functools.partial(_pre_body, ce, nblk, cap)
    return pl.kernel(
        body, out_type=out_type, mesh=mesh, scratch_types=scratch,
        compiler_params=pltpu.CompilerParams(needs_layout_passes=False),
    )(src_p, dst_p)


def _agg_body(d, cap,
              h_hbm, counts_hbm, srcl_hbm, dstl_hbm, zeros_hbm,
              out_hbm,
              acc, cnts_v, idx_s, idx_d, rows, gsem, ssem):
    c = lax.axis_index("c")
    s = lax.axis_index("s")
    pltpu.sync_copy(counts_hbm, cnts_v)

    for p in range(NRANGES // NC):
        r = c * (NRANGES // NC) + p

        for k in range(RACC // (NS * ZROWS)):
            off = pl.multiple_of(s * (RACC // NS) + k * ZROWS, 8)
            pltpu.sync_copy(zeros_hbm, acc.at[pl.ds(off, ZROWS)])
        plsc.subcore_barrier()

        iota = lax.iota(jnp.int32, LANES)
        for q in range(NW // NS):
            w2 = s * (NW // NS) + q
            crow = cnts_v[w2, pl.ds(0, LANES)]
            nb = jnp.sum(jnp.where(iota == r, crow, 0)) // BLK

            def idx_load(b, par):
                boff = pl.multiple_of(b * BLK, 8)
                pltpu.sync_copy(srcl_hbm.at[w2, r, pl.ds(boff, BLK)],
                                idx_s.at[par])
                pltpu.sync_copy(dstl_hbm.at[w2, r, pl.ds(boff, BLK)],
                                idx_d.at[par])

            def gather_start(par):
                pltpu.async_copy(h_hbm.at[idx_s.at[par]], rows.at[par], gsem)

            def gather_wait(par):
                pltpu.make_async_copy(h_hbm.at[pl.ds(0, BLK)],
                                      rows.at[par], gsem).wait()

            def scat_start(par):
                pltpu.async_copy(rows.at[par], acc.at[idx_d.at[par]], ssem,
                                 add=True)

            def scat_wait(par):
                pltpu.make_async_copy(h_hbm.at[pl.ds(0, BLK)],
                                      rows.at[par], ssem).wait()

            @pl.when(nb > 0)
            def _():
                idx_load(0, 0)
                gather_start(0)

            def step(b, par):
                nxt = 1 - par

                @pl.when(b + 1 < nb)
                def _():
                    @pl.when(b >= 1)
                    def _():
                        scat_wait(nxt)
                    idx_load(b + 1, nxt)
                    gather_start(nxt)

                gather_wait(par)
                scat_start(par)

            def pair_body(k, _):
                b0 = 2 * k

                @pl.when(b0 < nb)
                def _():
                    step(b0, 0)

                @pl.when(b0 + 1 < nb)
                def _():
                    step(b0 + 1, 1)
                return 0

            lax.fori_loop(0, (nb + 1) // 2, pair_body, 0)

            @pl.when(nb >= 2)
            def _():
                scat_wait(0)

            @pl.when(nb >= 1)
            def _():
                scat_wait(1)
        plsc.subcore_barrier()

        off = pl.multiple_of(s * WSPAN, 8)
        goff = pl.multiple_of(r * R + s * WSPAN, 8)
        pltpu.sync_copy(acc.at[pl.ds(off, WSPAN)],
                        out_hbm.at[pl.ds(goff, WSPAN)])
        plsc.subcore_barrier()


def _aggregate(h, counts, srcl, dstl, zeros, d, cap):
    mesh = plsc.VectorSubcoreMesh(core_axis_name="c", subcore_axis_name="s",
                                  num_cores=NC, num_subcores=NS)
    out_type = jax.ShapeDtypeStruct((NPAD, d), jnp.float32)
    scratch = [
        pltpu.VMEM_SHARED((RACC, d), jnp.float32),
        pltpu.VMEM((NW, LANES), jnp.int32),
        pltpu.VMEM((2, BLK), jnp.int32),
        pltpu.VMEM((2, BLK), jnp.int32),
        pltpu.VMEM((2, BLK, d), jnp.float32),
        pltpu.SemaphoreType.DMA,
        pltpu.SemaphoreType.DMA,
    ]
    body = functools.partial(_agg_body, d, cap)
    return pl.kernel(
        body, out_type=out_type, mesh=mesh, scratch_types=scratch,
        compiler_params=pltpu.CompilerParams(needs_layout_passes=False),
    )(h, counts, srcl, dstl, zeros)


def _dot(a, w):
    return jnp.dot(a.astype(jnp.bfloat16), w.astype(jnp.bfloat16),
                   preferred_element_type=jnp.float32)


def _mlp1_body(eps_ref, x_ref, agg_ref, w1a_ref, b1a_ref, w1b_ref, b1b_ref,
               s1_ref, t1_ref, out_ref):
    e = eps_ref[0, 0]
    z = (1.0 + e) * x_ref[...] + agg_ref[...][:, :64]
    h = jnp.maximum(_dot(z, w1a_ref[...]) + b1a_ref[...], 0.0)
    h = jnp.maximum(_dot(h, w1b_ref[...]) + b1b_ref[...], 0.0)
    out_ref[...] = h * s1_ref[...] + t1_ref[...]


def _mlp2_body(eps_ref, h_ref, agg_ref, w2a_ref, b2a_ref, s2_ref, t2_ref,
               out_ref):
    e = eps_ref[0, 0]
    z = (1.0 + e) * h_ref[...] + agg_ref[...]
    h = jnp.maximum(_dot(z, w2a_ref[...]) + b2a_ref[...], 0.0)
    out_ref[...] = h * s2_ref[...] + t2_ref[...]


def _mlp3_body(eps_ref, h2_ref, agg_ref, w3a_ref, b3a_ref, w3b_ref, b3b_ref,
               s3_ref, t3_ref, l1w_ref, l1b_ref, fcw_ref, fcb_ref, out_ref):
    e = eps_ref[0, 0]
    z = (1.0 + e) * h2_ref[...] + agg_ref[...]
    a = jnp.maximum(_dot(z, w3a_ref[...]) + b3a_ref[...], 0.0)
    a = jnp.maximum(_dot(a, w3b_ref[...]) + b3b_ref[...], 0.0)
    a = a * s3_ref[...] + t3_ref[...]
    d = jnp.maximum(_dot(a, l1w_ref[...]) + l1b_ref[...], 0.0)
    db = d.astype(jnp.bfloat16).astype(jnp.float32)
    fb = fcw_ref[...].astype(jnp.bfloat16).astype(jnp.float32)
    o = jnp.sum(db * fb, axis=1) + fcb_ref[0, 0]
    out_ref[...] = jnp.broadcast_to(o[None, :], (8, ROWBLK))


def _row_spec(d):
    return pl.BlockSpec((ROWBLK, d), lambda i: (i, 0))


def _full_spec(shape):
    return pl.BlockSpec(shape, lambda i: (0,) * len(shape))


def _mlp1(eps1, x, agg0, w1a, b1a, w1b, b1b, s1, t1):
    grid = (NPAD // ROWBLK,)
    return pl.pallas_call(
        _mlp1_body,
        grid=grid,
        in_specs=[_full_spec((1, 1)), _row_spec(64), _row_spec(128),
                  _full_spec((64, 128)), _full_spec((1, 128)),
                  _full_spec((128, 128)), _full_spec((1, 128)),
                  _full_spec((1, 128)), _full_spec((1, 128))],
        out_specs=_row_spec(128),
        out_shape=jax.ShapeDtypeStruct((NPAD, 128), jnp.float32),
    )(eps1, x, agg0, w1a, b1a, w1b, b1b, s1, t1)


def _mlp2(eps2, h, agg1, w2a, b2a, s2, t2):
    grid = (NPAD // ROWBLK,)
    return pl.pallas_call(
        _mlp2_body,
        grid=grid,
        in_specs=[_full_spec((1, 1)), _row_spec(128), _row_spec(128),
                  _full_spec((128, 128)), _full_spec((1, 128)),
                  _full_spec((1, 128)), _full_spec((1, 128))],
        out_specs=_row_spec(128),
        out_shape=jax.ShapeDtypeStruct((NPAD, 128), jnp.float32),
    )(eps2, h, agg1, w2a, b2a, s2, t2)


def _mlp3(eps3, h2, agg2, w3a, b3a, w3b, b3b, s3, t3, l1w, l1b, fcw, fcb):
    grid = (NPAD // ROWBLK,)
    return pl.pallas_call(
        _mlp3_body,
        grid=grid,
        in_specs=[_full_spec((1, 1)), _row_spec(128), _row_spec(128),
                  _full_spec((128, 128)), _full_spec((1, 128)),
                  _full_spec((128, 128)), _full_spec((1, 128)),
                  _full_spec((1, 128)), _full_spec((1, 128)),
                  _full_spec((128, 128)), _full_spec((1, 128)),
                  _full_spec((1, 128)), _full_spec((1, 1))],
        out_specs=pl.BlockSpec((8, ROWBLK), lambda i: (0, i)),
        out_shape=jax.ShapeDtypeStruct((8, NPAD), jnp.float32),
    )(eps3, h2, agg2, w3a, b3a, w3b, b3b, s3, t3, l1w, l1b, fcw, fcb)


def kernel(x, edge_index, eps1, eps2, eps3, W1a, b1a, W1b, b1b, bn1_g, bn1_b,
           W2a, b2a, bn2_g, bn2_b, W3a, b3a, W3b, b3b, bn3_g, bn3_b,
           lin1_W, lin1_b, fc_W, fc_b):
    n, din = x.shape
    e = edge_index.shape[1]
    ce = _cdiv(_cdiv(e, NW), BE) * BE
    epad = ce * NW
    cap = _cdiv(ce, BLK) * BLK

    src = edge_index[0]
    dst = edge_index[1]
    pad = epad - e
    src_p = jnp.concatenate([src, jnp.zeros((pad,), jnp.int32)])
    dst_p = jnp.concatenate([dst, jnp.full((pad,), jnp.int32(1 << 20))])
    x_p = jnp.pad(x, ((0, NPAD - n), (0, 0)))
    x_wide = jnp.pad(x, ((0, NPAD - n), (0, 128 - din)))

    counts, srcl, dstl = _preprocess(src_p, dst_p, ce, cap)

    inv = 1.0 / jnp.sqrt(1.0 + 1e-5)
    s1 = (bn1_g * inv)[None, :]; t1 = bn1_b[None, :]
    s2 = (bn2_g * inv)[None, :]; t2 = bn2_b[None, :]
    s3 = (bn3_g * inv)[None, :]; t3 = bn3_b[None, :]
    e1 = eps1.reshape(1, 1); e2 = eps2.reshape(1, 1); e3 = eps3.reshape(1, 1)

    zeros128 = jnp.zeros((ZROWS, 128), jnp.float32)
    agg0 = _aggregate(x_wide, counts, srcl, dstl, zeros128, 128, cap)
    h = _mlp1(e1, x_p, agg0, W1a, b1a[None, :], W1b, b1b[None, :], s1, t1)
    agg1 = _aggregate(h, counts, srcl, dstl, zeros128, 128, cap)
    h2 = _mlp2(e2, h, agg1, W2a, b2a[None, :], s2, t2)
    agg2 = _aggregate(h2, counts, srcl, dstl, zeros128, 128, cap)
    out2d = _mlp3(e3, h2, agg2, W3a, b3a[None, :], W3b, b3b[None, :], s3, t3,
                  lin1_W, lin1_b[None, :], fc_W.reshape(1, -1),
                  fc_b.reshape(1, 1))
    return out2d[0, :n]


# speedup vs baseline: 3.9653x; 1.0584x over previous
import functools

import jax
import jax.numpy as jnp
from jax import lax
from jax.experimental import pallas as pl
from jax.experimental.pallas import tpu as pltpu
from jax.experimental.pallas import tpu_sc as plsc

NC = 2
NS = 16
LANES = 16
NW = NC * NS

N = 50000
NRANGES = 6
R = 8704
RACC = R + 256
NPAD = NRANGES * R
WSPAN = R // NS
ZROWS = 280

BLK = 128
SLAB = 16
BE = 3200
NGROUPS = BE // LANES
STG = BLK + LANES

ROWBLK = 512


def _cdiv(a, b):
    return -(-a // b)


def _pre_body(ce, nblk, cap,
              src_hbm, dst_hbm,
              counts_hbm, srcl_hbm, dstl_hbm,
              src_blk, dst_blk, cnt_v, *stages):
    c = lax.axis_index("c")
    s = lax.axis_index("s")
    w = s * NC + c
    base = w * ce
    st_s = stages[:NRANGES]
    st_d = stages[NRANGES:]
    iota = lax.iota(jnp.int32, LANES)

    def group_body(g, carry):
        curs, wrs = carry
        off = pl.multiple_of(g * LANES, LANES)
        dv = dst_blk[pl.ds(off, LANES)]
        sv = src_blk[pl.ds(off, LANES)]
        new_curs = []
        new_wrs = []
        for r in range(NRANGES):
            cur = curs[r]
            wr = wrs[r]
            m = (dv >= r * R) & (dv < (r + 1) * R)
            mi = m.astype(jnp.int32)
            pos = cur + plsc.cumsum(mi) - 1
            plsc.store_scatter(st_s[r], [pos], sv, mask=m)
            plsc.store_scatter(st_d[r], [pos], dv - (r * R), mask=m)
            cur = cur + jnp.sum(mi)

            def flush(args):
                cur_i, wr_i = args
                wrm = pl.multiple_of(wr_i, BLK)
                pltpu.sync_copy(st_s[r].at[pl.ds(0, BLK)],
                                srcl_hbm.at[w, r, pl.ds(wrm, BLK)])
                pltpu.sync_copy(st_d[r].at[pl.ds(0, BLK)],
                                dstl_hbm.at[w, r, pl.ds(wrm, BLK)])
                tail_s = st_s[r][pl.ds(BLK, LANES)]
                tail_d = st_d[r][pl.ds(BLK, LANES)]
                st_s[r][pl.ds(0, LANES)] = tail_s
                st_d[r][pl.ds(0, LANES)] = tail_d
                return cur_i - BLK, wr_i + BLK

            cur, wr = lax.cond(cur >= BLK, flush, lambda a: a, (cur, wr))
            new_curs.append(cur)
            new_wrs.append(wr)
        return tuple(new_curs), tuple(new_wrs)

    def blk_body(b, carry):
        boff = pl.multiple_of(base + b * BE, BE)
        pltpu.sync_copy(src_hbm.at[pl.ds(boff, BE)], src_blk)
        pltpu.sync_copy(dst_hbm.at[pl.ds(boff, BE)], dst_blk)
        return lax.fori_loop(0, NGROUPS, group_body, carry)

    zero4 = (jnp.int32(0),) * NRANGES
    curs, wrs = lax.fori_loop(0, nblk, blk_body, (zero4, zero4))

    cnt_vec = jnp.zeros((LANES,), jnp.int32)
    for r in range(NRANGES):
        cur = curs[r]
        wr = wrs[r]
        for g in range(BLK // LANES):
            lanes_g = g * LANES + iota
            keep = lanes_g < cur
            sv = st_s[r][pl.ds(g * LANES, LANES)]
            dvv = st_d[r][pl.ds(g * LANES, LANES)]
            st_s[r][pl.ds(g * LANES, LANES)] = jnp.where(keep, sv, 0)
            st_d[r][pl.ds(g * LANES, LANES)] = jnp.where(keep, dvv, R + iota)

        @pl.when(cur > 0)
        def _():
            wrm = pl.multiple_of(wr, BLK)
            pltpu.sync_copy(st_s[r].at[pl.ds(0, BLK)],
                            srcl_hbm.at[w, r, pl.ds(wrm, BLK)])
            pltpu.sync_copy(st_d[r].at[pl.ds(0, BLK)],
                            dstl_hbm.at[w, r, pl.ds(wrm, BLK)])

        total = wr + jnp.where(cur > 0, BLK, 0)
        cnt_vec = jnp.where(iota == r, total, cnt_vec)

    cnt_v[pl.ds(0, LANES)] = cnt_vec
    pltpu.sync_copy(cnt_v, counts_hbm.at[w])


def _preprocess(src_p, dst_p, ce, cap):
    nblk = ce // BE
    mesh = plsc.VectorSubcoreMesh(core_axis_name="c", subcore_axis_name="s",
                                  num_cores=NC, num_subcores=NS)
    out_type = [
        jax.ShapeDtypeStruct((NW, LANES), jnp.int32),
        jax.ShapeDtypeStruct((NW, NRANGES, cap), jnp.int32),
        jax.ShapeDtypeStruct((NW, NRANGES, cap), jnp.int32),
    ]
    scratch = ([pltpu.VMEM((BE,), jnp.int32)] * 2
               + [pltpu.VMEM((LANES,), jnp.int32)]
               + [pltpu.VMEM((STG,), jnp.int32)] * (2 * NRANGES))
    body = functools.partial(_pre_body, ce, nblk, cap)
    return pl.kernel(
        body, out_type=out_type, mesh=mesh, scratch_types=scratch,
        compiler_params=pltpu.CompilerParams(needs_layout_passes=False),
    )(src_p, dst_p)


def _agg_body(d, cap,
              h_hbm, counts_hbm, srcl_hbm, dstl_hbm, zeros_hbm,
              out_hbm,
              acc, cnts_v, idx_s, idx_d, rows, gsem, ssem):
    c = lax.axis_index("c")
    s = lax.axis_index("s")
    pltpu.sync_copy(counts_hbm, cnts_v)

    for p in range(NRANGES // NC):
        r = c * (NRANGES // NC) + p

        for k in range(RACC // (NS * ZROWS)):
            off = pl.multiple_of(s * (RACC // NS) + k * ZROWS, 8)
            pltpu.sync_copy(zeros_hbm, acc.at[pl.ds(off, ZROWS)])
        plsc.subcore_barrier()

        iota = lax.iota(jnp.int32, LANES)
        for q in range(NW // NS):
            w2 = s * (NW // NS) + q
            crow = cnts_v[w2, pl.ds(0, LANES)]
            nb = jnp.sum(jnp.where(iota == r, crow, 0)) // BLK

            def idx_slab_load(b):
                sb = b // SLAB
                roff = pl.multiple_of((sb % 2) * SLAB, SLAB)
                soff = pl.multiple_of(sb * SLAB, SLAB)
                pltpu.sync_copy(srcl_hbm.at[w2, r, pl.ds(soff, SLAB)],
                                idx_s.at[pl.ds(roff, SLAB)])
                pltpu.sync_copy(dstl_hbm.at[w2, r, pl.ds(soff, SLAB)],
                                idx_d.at[pl.ds(roff, SLAB)])

            def gather_start(b, par):
                pltpu.async_copy(h_hbm.at[idx_s.at[b % (2 * SLAB)]],
                                 rows.at[par], gsem)

            def gather_wait(par):
                pltpu.make_async_copy(h_hbm.at[pl.ds(0, BLK)],
                                      rows.at[par], gsem).wait()

            def scat_start(b, par):
                pltpu.async_copy(rows.at[par],
                                 acc.at[idx_d.at[b % (2 * SLAB)]], ssem,
                                 add=True)

            def scat_wait():
                pltpu.make_async_copy(h_hbm.at[pl.ds(0, BLK)],
                                      rows.at[0], ssem).wait()

            @pl.when(nb > 0)
            def _():
                idx_slab_load(0)
                gather_start(0, 0)

            def step(b, par):
                nxt = 1 - par

                @pl.when(b + 1 < nb)
                def _():
                    @pl.when(b >= 1)
                    def _():
                        scat_wait()

                    @pl.when((b + 1) % SLAB == 0)
                    def _():
                        idx_slab_load(b + 1)

                    gather_start(b + 1, nxt)

                gather_wait(par)
                scat_start(b, par)

            def pair_body(k, _):
                b0 = 2 * k

                @pl.when(b0 < nb)
                def _():
                    step(b0, 0)

                @pl.when(b0 + 1 < nb)
                def _():
                    step(b0 + 1, 1)
                return 0

            lax.fori_loop(0, (nb + 1) // 2, pair_body, 0)

            @pl.when(nb >= 2)
            def _():
                scat_wait()

            @pl.when(nb >= 1)
            def _():
                scat_wait()
        plsc.subcore_barrier()

        off = pl.multiple_of(s * WSPAN, 8)
        goff = pl.multiple_of(r * R + s * WSPAN, 8)
        pltpu.sync_copy(acc.at[pl.ds(off, WSPAN)],
                        out_hbm.at[pl.ds(goff, WSPAN)])
        plsc.subcore_barrier()


def _aggregate(h, counts, srcl, dstl, zeros, d, cap):
    mesh = plsc.VectorSubcoreMesh(core_axis_name="c", subcore_axis_name="s",
                                  num_cores=NC, num_subcores=NS)
    out_type = jax.ShapeDtypeStruct((NPAD, d), jnp.float32)
    scratch = [
        pltpu.VMEM_SHARED((RACC, d), jnp.float32),
        pltpu.VMEM((NW, LANES), jnp.int32),
        pltpu.VMEM((2 * SLAB, BLK), jnp.int32),
        pltpu.VMEM((2 * SLAB, BLK), jnp.int32),
        pltpu.VMEM((2, BLK, d), jnp.float32),
        pltpu.SemaphoreType.DMA,
        pltpu.SemaphoreType.DMA,
    ]
    body = functools.partial(_agg_body, d, cap)
    return pl.kernel(
        body, out_type=out_type, mesh=mesh, scratch_types=scratch,
        compiler_params=pltpu.CompilerParams(needs_layout_passes=False),
    )(h, counts, srcl, dstl, zeros)


def _dot(a, w):
    return jnp.dot(a.astype(jnp.bfloat16), w.astype(jnp.bfloat16),
                   preferred_element_type=jnp.float32)


def _mlp1_body(eps_ref, x_ref, agg_ref, w1a_ref, b1a_ref, w1b_ref, b1b_ref,
               s1_ref, t1_ref, out_ref):
    e = eps_ref[0, 0]
    z = (1.0 + e) * x_ref[...] + agg_ref[...][:, :64]
    h = jnp.maximum(_dot(z, w1a_ref[...]) + b1a_ref[...], 0.0)
    h = jnp.maximum(_dot(h, w1b_ref[...]) + b1b_ref[...], 0.0)
    out_ref[...] = h * s1_ref[...] + t1_ref[...]


def _mlp2_body(eps_ref, h_ref, agg_ref, w2a_ref, b2a_ref, s2_ref, t2_ref,
               out_ref):
    e = eps_ref[0, 0]
    z = (1.0 + e) * h_ref[...] + agg_ref[...]
    h = jnp.maximum(_dot(z, w2a_ref[...]) + b2a_ref[...], 0.0)
    out_ref[...] = h * s2_ref[...] + t2_ref[...]


def _mlp3_body(eps_ref, h2_ref, agg_ref, w3a_ref, b3a_ref, w3b_ref, b3b_ref,
               s3_ref, t3_ref, l1w_ref, l1b_ref, fcw_ref, fcb_ref, out_ref):
    e = eps_ref[0, 0]
    z = (1.0 + e) * h2_ref[...] + agg_ref[...]
    a = jnp.maximum(_dot(z, w3a_ref[...]) + b3a_ref[...], 0.0)
    a = jnp.maximum(_dot(a, w3b_ref[...]) + b3b_ref[...], 0.0)
    a = a * s3_ref[...] + t3_ref[...]
    d = jnp.maximum(_dot(a, l1w_ref[...]) + l1b_ref[...], 0.0)
    db = d.astype(jnp.bfloat16).astype(jnp.float32)
    fb = fcw_ref[...].astype(jnp.bfloat16).astype(jnp.float32)
    o = jnp.sum(db * fb, axis=1) + fcb_ref[0, 0]
    out_ref[...] = jnp.broadcast_to(o[None, :], (8, ROWBLK))


def _row_spec(d):
    return pl.BlockSpec((ROWBLK, d), lambda i: (i, 0))


def _full_spec(shape):
    return pl.BlockSpec(shape, lambda i: (0,) * len(shape))


def _mlp1(eps1, x, agg0, w1a, b1a, w1b, b1b, s1, t1):
    grid = (NPAD // ROWBLK,)
    return pl.pallas_call(
        _mlp1_body,
        grid=grid,
        in_specs=[_full_spec((1, 1)), _row_spec(64), _row_spec(128),
                  _full_spec((64, 128)), _full_spec((1, 128)),
                  _full_spec((128, 128)), _full_spec((1, 128)),
                  _full_spec((1, 128)), _full_spec((1, 128))],
        out_specs=_row_spec(128),
        out_shape=jax.ShapeDtypeStruct((NPAD, 128), jnp.float32),
    )(eps1, x, agg0, w1a, b1a, w1b, b1b, s1, t1)


def _mlp2(eps2, h, agg1, w2a, b2a, s2, t2):
    grid = (NPAD // ROWBLK,)
    return pl.pallas_call(
        _mlp2_body,
        grid=grid,
        in_specs=[_full_spec((1, 1)), _row_spec(128), _row_spec(128),
                  _full_spec((128, 128)), _full_spec((1, 128)),
                  _full_spec((1, 128)), _full_spec((1, 128))],
        out_specs=_row_spec(128),
        out_shape=jax.ShapeDtypeStruct((NPAD, 128), jnp.float32),
    )(eps2, h, agg1, w2a, b2a, s2, t2)


def _mlp3(eps3, h2, agg2, w3a, b3a, w3b, b3b, s3, t3, l1w, l1b, fcw, fcb):
    grid = (NPAD // ROWBLK,)
    return pl.pallas_call(
        _mlp3_body,
        grid=grid,
        in_specs=[_full_spec((1, 1)), _row_spec(128), _row_spec(128),
                  _full_spec((128, 128)), _full_spec((1, 128)),
                  _full_spec((128, 128)), _full_spec((1, 128)),
                  _full_spec((1, 128)), _full_spec((1, 128)),
                  _full_spec((128, 128)), _full_spec((1, 128)),
                  _full_spec((1, 128)), _full_spec((1, 1))],
        out_specs=pl.BlockSpec((8, ROWBLK), lambda i: (0, i)),
        out_shape=jax.ShapeDtypeStruct((8, NPAD), jnp.float32),
    )(eps3, h2, agg2, w3a, b3a, w3b, b3b, s3, t3, l1w, l1b, fcw, fcb)


def kernel(x, edge_index, eps1, eps2, eps3, W1a, b1a, W1b, b1b, bn1_g, bn1_b,
           W2a, b2a, bn2_g, bn2_b, W3a, b3a, W3b, b3b, bn3_g, bn3_b,
           lin1_W, lin1_b, fc_W, fc_b):
    n, din = x.shape
    e = edge_index.shape[1]
    ce = _cdiv(_cdiv(e, NW), BE) * BE
    epad = ce * NW
    cap = _cdiv(ce, BLK * SLAB) * (BLK * SLAB)

    src = edge_index[0]
    dst = edge_index[1]
    pad = epad - e
    src_p = jnp.concatenate([src, jnp.zeros((pad,), jnp.int32)])
    dst_p = jnp.concatenate([dst, jnp.full((pad,), jnp.int32(1 << 20))])
    x_p = jnp.pad(x, ((0, NPAD - n), (0, 0)))
    x_wide = jnp.pad(x, ((0, NPAD - n), (0, 128 - din)))

    counts, srcl, dstl = _preprocess(src_p, dst_p, ce, cap)
    srcl = srcl.reshape(NW, NRANGES, cap // BLK, BLK)
    dstl = dstl.reshape(NW, NRANGES, cap // BLK, BLK)

    inv = 1.0 / jnp.sqrt(1.0 + 1e-5)
    s1 = (bn1_g * inv)[None, :]; t1 = bn1_b[None, :]
    s2 = (bn2_g * inv)[None, :]; t2 = bn2_b[None, :]
    s3 = (bn3_g * inv)[None, :]; t3 = bn3_b[None, :]
    e1 = eps1.reshape(1, 1); e2 = eps2.reshape(1, 1); e3 = eps3.reshape(1, 1)

    zeros128 = jnp.zeros((ZROWS, 128), jnp.float32)
    agg0 = _aggregate(x_wide, counts, srcl, dstl, zeros128, 128, cap)
    h = _mlp1(e1, x_p, agg0, W1a, b1a[None, :], W1b, b1b[None, :], s1, t1)
    agg1 = _aggregate(h, counts, srcl, dstl, zeros128, 128, cap)
    h2 = _mlp2(e2, h, agg1, W2a, b2a[None, :], s2, t2)
    agg2 = _aggregate(h2, counts, srcl, dstl, zeros128, 128, cap)
    out2d = _mlp3(e3, h2, agg2, W3a, b3a[None, :], W3b, b3b[None, :], s3, t3,
                  lin1_W, lin1_b[None, :], fc_W.reshape(1, -1),
                  fc_b.reshape(1, 1))
    return out2d[0, :n]


# speedup vs baseline: 4.0149x; 1.0125x over previous
import functools

import jax
import jax.numpy as jnp
from jax import lax
from jax.experimental import pallas as pl
from jax.experimental.pallas import tpu as pltpu
from jax.experimental.pallas import tpu_sc as plsc

NC = 2
NS = 16
LANES = 16
NW = NC * NS

N = 50000
NRANGES = 6
R = 8704
RACC = R + 256
NPAD = NRANGES * R
WSPAN = R // NS
ZROWS = 280

BLK = 128
SLAB = 8
BE = 3200
NGROUPS = BE // LANES
STG = BLK + LANES

ROWBLK = 512


def _cdiv(a, b):
    return -(-a // b)


def _pre_body(ce, nblk, cap,
              src_hbm, dst_hbm,
              counts_hbm, srcl_hbm, dstl_hbm,
              src_blk, dst_blk, cnt_v, *stages):
    c = lax.axis_index("c")
    s = lax.axis_index("s")
    w = s * NC + c
    base = w * ce
    st_s = stages[:NRANGES]
    st_d = stages[NRANGES:]
    iota = lax.iota(jnp.int32, LANES)

    def group_body(g, carry):
        curs, wrs = carry
        off = pl.multiple_of(g * LANES, LANES)
        dv = dst_blk[pl.ds(off, LANES)]
        sv = src_blk[pl.ds(off, LANES)]
        new_curs = []
        new_wrs = []
        for r in range(NRANGES):
            cur = curs[r]
            wr = wrs[r]
            m = (dv >= r * R) & (dv < (r + 1) * R)
            mi = m.astype(jnp.int32)
            pos = cur + plsc.cumsum(mi) - 1
            plsc.store_scatter(st_s[r], [pos], sv, mask=m)
            plsc.store_scatter(st_d[r], [pos], dv - (r * R), mask=m)
            cur = cur + jnp.sum(mi)

            def flush(args):
                cur_i, wr_i = args
                wrm = pl.multiple_of(wr_i, BLK)
                pltpu.sync_copy(st_s[r].at[pl.ds(0, BLK)],
                                srcl_hbm.at[w, r, pl.ds(wrm, BLK)])
                pltpu.sync_copy(st_d[r].at[pl.ds(0, BLK)],
                                dstl_hbm.at[w, r, pl.ds(wrm, BLK)])
                tail_s = st_s[r][pl.ds(BLK, LANES)]
                tail_d = st_d[r][pl.ds(BLK, LANES)]
                st_s[r][pl.ds(0, LANES)] = tail_s
                st_d[r][pl.ds(0, LANES)] = tail_d
                return cur_i - BLK, wr_i + BLK

            cur, wr = lax.cond(cur >= BLK, flush, lambda a: a, (cur, wr))
            new_curs.append(cur)
            new_wrs.append(wr)
        return tuple(new_curs), tuple(new_wrs)

    def blk_body(b, carry):
        boff = pl.multiple_of(base + b * BE, BE)
        pltpu.sync_copy(src_hbm.at[pl.ds(boff, BE)], src_blk)
        pltpu.sync_copy(dst_hbm.at[pl.ds(boff, BE)], dst_blk)
        return lax.fori_loop(0, NGROUPS, group_body, carry)

    zero4 = (jnp.int32(0),) * NRANGES
    curs, wrs = lax.fori_loop(0, nblk, blk_body, (zero4, zero4))

    cnt_vec = jnp.zeros((LANES,), jnp.int32)
    for r in range(NRANGES):
        cur = curs[r]
        wr = wrs[r]
        for g in range(BLK // LANES):
            lanes_g = g * LANES + iota
            keep = lanes_g < cur
            sv = st_s[r][pl.ds(g * LANES, LANES)]
            dvv = st_d[r][pl.ds(g * LANES, LANES)]
            st_s[r][pl.ds(g * LANES, LANES)] = jnp.where(keep, sv, 0)
            st_d[r][pl.ds(g * LANES, LANES)] = jnp.where(keep, dvv, R + iota)

        @pl.when(cur > 0)
        def _():
            wrm = pl.multiple_of(wr, BLK)
            pltpu.sync_copy(st_s[r].at[pl.ds(0, BLK)],
                            srcl_hbm.at[w, r, pl.ds(wrm, BLK)])
            pltpu.sync_copy(st_d[r].at[pl.ds(0, BLK)],
                            dstl_hbm.at[w, r, pl.ds(wrm, BLK)])

        total = wr + jnp.where(cur > 0, BLK, 0)
        cnt_vec = jnp.where(iota == r, total, cnt_vec)

    cnt_v[pl.ds(0, LANES)] = cnt_vec
    pltpu.sync_copy(cnt_v, counts_hbm.at[w])


def _preprocess(src_p, dst_p, ce, cap):
    nblk = ce // BE
    mesh = plsc.VectorSubcoreMesh(core_axis_name="c", subcore_axis_name="s",
                                  num_cores=NC, num_subcores=NS)
    out_type = [
        jax.ShapeDtypeStruct((NW, LANES), jnp.int32),
        jax.ShapeDtypeStruct((NW, NRANGES, cap), jnp.int32),
        jax.ShapeDtypeStruct((NW, NRANGES, cap), jnp.int32),
    ]
    scratch = ([pltpu.VMEM((BE,), jnp.int32)] * 2
               + [pltpu.VMEM((LANES,), jnp.int32)]
               + [pltpu.VMEM((STG,), jnp.int32)] * (2 * NRANGES))
    body = functools.partial(_pre_body, ce, nblk, cap)
    return pl.kernel(
        body, out_type=out_type, mesh=mesh, scratch_types=scratch,
        compiler_params=pltpu.CompilerParams(needs_layout_passes=False),
    )(src_p, dst_p)


def _agg_body(d, cap,
              h_hbm, counts_hbm, srcl_hbm, dstl_hbm, zeros_hbm,
              out_hbm,
              acc, cnts_v, idx_s, idx_d, rows, gsem, ssem):
    c = lax.axis_index("c")
    s = lax.axis_index("s")
    pltpu.sync_copy(counts_hbm, cnts_v)

    for p in range(NRANGES // NC):
        r = c * (NRANGES // NC) + p

        for k in range(RACC // (NS * ZROWS)):
            off = pl.multiple_of(s * (RACC // NS) + k * ZROWS, 8)
            pltpu.async_copy(zeros_hbm, acc.at[pl.ds(off, ZROWS)], gsem)
        for k in range(RACC // (NS * ZROWS)):
            off = pl.multiple_of(s * (RACC // NS) + k * ZROWS, 8)
            pltpu.make_async_copy(zeros_hbm, acc.at[pl.ds(off, ZROWS)],
                                  gsem).wait()
        plsc.subcore_barrier()

        iota = lax.iota(jnp.int32, LANES)
        for q in range(NW // NS):
            w2 = s * (NW // NS) + q
            crow = cnts_v[w2, pl.ds(0, LANES)]
            nb = jnp.sum(jnp.where(iota == r, crow, 0)) // BLK

            def idx_slab_load(b):
                sb = b // SLAB
                roff = pl.multiple_of((sb % 2) * SLAB, SLAB)
                soff = pl.multiple_of(sb * SLAB, SLAB)
                pltpu.sync_copy(srcl_hbm.at[w2, r, pl.ds(soff, SLAB)],
                                idx_s.at[pl.ds(roff, SLAB)])
                pltpu.sync_copy(dstl_hbm.at[w2, r, pl.ds(soff, SLAB)],
                                idx_d.at[pl.ds(roff, SLAB)])

            def gather_start(b, par):
                pltpu.async_copy(h_hbm.at[idx_s.at[b % (2 * SLAB)]],
                                 rows.at[par], gsem)

            def gather_wait(par):
                pltpu.make_async_copy(h_hbm.at[pl.ds(0, BLK)],
                                      rows.at[par], gsem).wait()

            def scat_start(b, par):
                pltpu.async_copy(rows.at[par],
                                 acc.at[idx_d.at[b % (2 * SLAB)]], ssem,
                                 add=True)

            def scat_wait():
                pltpu.make_async_copy(h_hbm.at[pl.ds(0, BLK)],
                                      rows.at[0], ssem).wait()

            @pl.when(nb > 0)
            def _():
                idx_slab_load(0)
                gather_start(0, 0)

            def step(b, par):
                nxt = (par + 1) % 3

                @pl.when(b + 1 < nb)
                def _():
                    @pl.when(b >= 2)
                    def _():
                        scat_wait()

                    @pl.when((b + 1) % SLAB == 0)
                    def _():
                        idx_slab_load(b + 1)

                    gather_start(b + 1, nxt)

                gather_wait(par)
                scat_start(b, par)

            def trio_body(k, _):
                b0 = 3 * k

                @pl.when(b0 < nb)
                def _():
                    step(b0, 0)

                @pl.when(b0 + 1 < nb)
                def _():
                    step(b0 + 1, 1)

                @pl.when(b0 + 2 < nb)
                def _():
                    step(b0 + 2, 2)
                return 0

            lax.fori_loop(0, (nb + 2) // 3, trio_body, 0)

            @pl.when(nb >= 2)
            def _():
                scat_wait()

            @pl.when(nb >= 1)
            def _():
                scat_wait()
        plsc.subcore_barrier()

        off = pl.multiple_of(s * WSPAN, 8)
        goff = pl.multiple_of(r * R + s * WSPAN, 8)
        pltpu.sync_copy(acc.at[pl.ds(off, WSPAN)],
                        out_hbm.at[pl.ds(goff, WSPAN)])
        plsc.subcore_barrier()


def _aggregate(h, counts, srcl, dstl, zeros, d, cap):
    mesh = plsc.VectorSubcoreMesh(core_axis_name="c", subcore_axis_name="s",
                                  num_cores=NC, num_subcores=NS)
    out_type = jax.ShapeDtypeStruct((NPAD, d), jnp.float32)
    scratch = [
        pltpu.VMEM_SHARED((RACC, d), jnp.float32),
        pltpu.VMEM((NW, LANES), jnp.int32),
        pltpu.VMEM((2 * SLAB, BLK), jnp.int32),
        pltpu.VMEM((2 * SLAB, BLK), jnp.int32),
        pltpu.VMEM((3, BLK, d), jnp.float32),
        pltpu.SemaphoreType.DMA,
        pltpu.SemaphoreType.DMA,
    ]
    body = functools.partial(_agg_body, d, cap)
    return pl.kernel(
        body, out_type=out_type, mesh=mesh, scratch_types=scratch,
        compiler_params=pltpu.CompilerParams(needs_layout_passes=False),
    )(h, counts, srcl, dstl, zeros)


def _dot(a, w):
    return jnp.dot(a.astype(jnp.bfloat16), w.astype(jnp.bfloat16),
                   preferred_element_type=jnp.float32)


def _mlp1_body(eps_ref, x_ref, agg_ref, w1a_ref, b1a_ref, w1b_ref, b1b_ref,
               s1_ref, t1_ref, out_ref):
    e = eps_ref[0, 0]
    z = (1.0 + e) * x_ref[...] + agg_ref[...][:, :64]
    h = jnp.maximum(_dot(z, w1a_ref[...]) + b1a_ref[...], 0.0)
    h = jnp.maximum(_dot(h, w1b_ref[...]) + b1b_ref[...], 0.0)
    out_ref[...] = h * s1_ref[...] + t1_ref[...]


def _mlp2_body(eps_ref, h_ref, agg_ref, w2a_ref, b2a_ref, s2_ref, t2_ref,
               out_ref):
    e = eps_ref[0, 0]
    z = (1.0 + e) * h_ref[...] + agg_ref[...]
    h = jnp.maximum(_dot(z, w2a_ref[...]) + b2a_ref[...], 0.0)
    out_ref[...] = h * s2_ref[...] + t2_ref[...]


def _mlp3_body(eps_ref, h2_ref, agg_ref, w3a_ref, b3a_ref, w3b_ref, b3b_ref,
               s3_ref, t3_ref, l1w_ref, l1b_ref, fcw_ref, fcb_ref, out_ref):
    e = eps_ref[0, 0]
    z = (1.0 + e) * h2_ref[...] + agg_ref[...]
    a = jnp.maximum(_dot(z, w3a_ref[...]) + b3a_ref[...], 0.0)
    a = jnp.maximum(_dot(a, w3b_ref[...]) + b3b_ref[...], 0.0)
    a = a * s3_ref[...] + t3_ref[...]
    d = jnp.maximum(_dot(a, l1w_ref[...]) + l1b_ref[...], 0.0)
    db = d.astype(jnp.bfloat16).astype(jnp.float32)
    fb = fcw_ref[...].astype(jnp.bfloat16).astype(jnp.float32)
    o = jnp.sum(db * fb, axis=1) + fcb_ref[0, 0]
    out_ref[...] = jnp.broadcast_to(o[None, :], (8, ROWBLK))


def _row_spec(d):
    return pl.BlockSpec((ROWBLK, d), lambda i: (i, 0))


def _full_spec(shape):
    return pl.BlockSpec(shape, lambda i: (0,) * len(shape))


def _mlp1(eps1, x, agg0, w1a, b1a, w1b, b1b, s1, t1):
    grid = (NPAD // ROWBLK,)
    return pl.pallas_call(
        _mlp1_body,
        grid=grid,
        in_specs=[_full_spec((1, 1)), _row_spec(64), _row_spec(128),
                  _full_spec((64, 128)), _full_spec((1, 128)),
                  _full_spec((128, 128)), _full_spec((1, 128)),
                  _full_spec((1, 128)), _full_spec((1, 128))],
        out_specs=_row_spec(128),
        out_shape=jax.ShapeDtypeStruct((NPAD, 128), jnp.float32),
    )(eps1, x, agg0, w1a, b1a, w1b, b1b, s1, t1)


def _mlp2(eps2, h, agg1, w2a, b2a, s2, t2):
    grid = (NPAD // ROWBLK,)
    return pl.pallas_call(
        _mlp2_body,
        grid=grid,
        in_specs=[_full_spec((1, 1)), _row_spec(128), _row_spec(128),
                  _full_spec((128, 128)), _full_spec((1, 128)),
                  _full_spec((1, 128)), _full_spec((1, 128))],
        out_specs=_row_spec(128),
        out_shape=jax.ShapeDtypeStruct((NPAD, 128), jnp.float32),
    )(eps2, h, agg1, w2a, b2a, s2, t2)


def _mlp3(eps3, h2, agg2, w3a, b3a, w3b, b3b, s3, t3, l1w, l1b, fcw, fcb):
    grid = (NPAD // ROWBLK,)
    return pl.pallas_call(
        _mlp3_body,
        grid=grid,
        in_specs=[_full_spec((1, 1)), _row_spec(128), _row_spec(128),
                  _full_spec((128, 128)), _full_spec((1, 128)),
                  _full_spec((128, 128)), _full_spec((1, 128)),
                  _full_spec((1, 128)), _full_spec((1, 128)),
                  _full_spec((128, 128)), _full_spec((1, 128)),
                  _full_spec((1, 128)), _full_spec((1, 1))],
        out_specs=pl.BlockSpec((8, ROWBLK), lambda i: (0, i)),
        out_shape=jax.ShapeDtypeStruct((8, NPAD), jnp.float32),
    )(eps3, h2, agg2, w3a, b3a, w3b, b3b, s3, t3, l1w, l1b, fcw, fcb)


def kernel(x, edge_index, eps1, eps2, eps3, W1a, b1a, W1b, b1b, bn1_g, bn1_b,
           W2a, b2a, bn2_g, bn2_b, W3a, b3a, W3b, b3b, bn3_g, bn3_b,
           lin1_W, lin1_b, fc_W, fc_b):
    n, din = x.shape
    e = edge_index.shape[1]
    ce = _cdiv(_cdiv(e, NW), BE) * BE
    epad = ce * NW
    cap = _cdiv(ce, BLK * SLAB) * (BLK * SLAB)

    src = edge_index[0]
    dst = edge_index[1]
    pad = epad - e
    src_p = jnp.concatenate([src, jnp.zeros((pad,), jnp.int32)])
    dst_p = jnp.concatenate([dst, jnp.full((pad,), jnp.int32(1 << 20))])
    x_p = jnp.pad(x, ((0, NPAD - n), (0, 0)))
    x_wide = jnp.pad(x, ((0, NPAD - n), (0, 128 - din)))

    counts, srcl, dstl = _preprocess(src_p, dst_p, ce, cap)
    srcl = srcl.reshape(NW, NRANGES, cap // BLK, BLK)
    dstl = dstl.reshape(NW, NRANGES, cap // BLK, BLK)

    inv = 1.0 / jnp.sqrt(1.0 + 1e-5)
    s1 = (bn1_g * inv)[None, :]; t1 = bn1_b[None, :]
    s2 = (bn2_g * inv)[None, :]; t2 = bn2_b[None, :]
    s3 = (bn3_g * inv)[None, :]; t3 = bn3_b[None, :]
    e1 = eps1.reshape(1, 1); e2 = eps2.reshape(1, 1); e3 = eps3.reshape(1, 1)

    zeros128 = jnp.zeros((ZROWS, 128), jnp.float32)
    agg0 = _aggregate(x_wide, counts, srcl, dstl, zeros128, 128, cap)
    h = _mlp1(e1, x_p, agg0, W1a, b1a[None, :], W1b, b1b[None, :], s1, t1)
    agg1 = _aggregate(h, counts, srcl, dstl, zeros128, 128, cap)
    h2 = _mlp2(e2, h, agg1, W2a, b2a[None, :], s2, t2)
    agg2 = _aggregate(h2, counts, srcl, dstl, zeros128, 128, cap)
    out2d = _mlp3(e3, h2, agg2, W3a, b3a[None, :], W3b, b3b[None, :], s3, t3,
                  lin1_W, lin1_b[None, :], fc_W.reshape(1, -1),
                  fc_b.reshape(1, 1))
    return out2d[0, :n]
